# baseline (speedup 1.0000x reference)
import jax
import jax.numpy as jnp
from jax import lax
from jax.experimental import pallas as pl
from jax.experimental.pallas import tpu as pltpu
from jax.experimental.pallas import tpu_sc as plsc

B = 4096
NAME_LEN = 20
D_ID = 8
D_NAME = 16
D_GICS = 8
D_OUT = D_ID + D_NAME + D_GICS
ID_VOCAB = 100001
GICS_VOCAB = 1001

NC, NS, L = 2, 16, 16
NW = NC * NS
BPW = B // NW
NGRP = BPW // L
RND = 4
NROUND = NAME_LEN // RND


def _treesum(vs):
    while len(vs) > 1:
        vs = [a + b for a, b in zip(vs[::2], vs[1::2])] + (
            [vs[-1]] if len(vs) % 2 else [])
    return vs[0]


def _body(emb_id_f, emb_name, emb_gics_f, idv, tok_t, gicsv, out_hbm,
          ididx_v, gicsidx_v, idix_v, tok_v, namerows_v, gicstab_v,
          nameacc_v, emb0_v, emb0T_v, z_v, rcnt_v, out_tv,
          semid, semg, sem2, semt, *semn):
    wid = lax.axis_index("c") * NS + lax.axis_index("s")
    base = wid * BPW
    iota = lax.iota(jnp.int32, L)

    import jax as _jax
    _s1 = _jax.named_scope("p1_stage"); _s1.__enter__()
    gics_stage = pltpu.async_copy(emb_gics_f, gicstab_v, semg)
    tok_stage = pltpu.async_copy(tok_t.at[:, pl.ds(base, BPW)], tok_v, semt)
    stage = [
        pltpu.async_copy(idv.at[pl.ds(base, BPW)], ididx_v, sem2),
        pltpu.async_copy(gicsv.at[pl.ds(base, BPW)], gicsidx_v, sem2),
        pltpu.async_copy(emb_name.at[pl.ds(0, 1)], emb0_v, sem2),
    ]

    tok_stage.wait()
    name_copies = [pltpu.async_copy(emb_name.at[tok_v.at[l]],
                                    namerows_v.at[pl.ds(l * BPW, BPW)],
                                    semn[l // RND])
                   for l in range(NAME_LEN)]

    for cp in stage:
        cp.wait()

    for d in range(D_ID):
        for c in range(NGRP):
            sl = pl.ds(c * L, L)
            idix_v[d, sl] = ididx_v[sl] + d * ID_VOCAB
    id_copies = [pltpu.async_copy(emb_id_f.at[idix_v.at[d]],
                                  out_tv.at[d], semid)
                 for d in range(D_ID)]

    e0vec = emb0_v[0, :]
    for c in range(L):
        plsc.store_scatter(emb0T_v, [iota, jnp.full((L,), c, jnp.int32)],
                           e0vec)

    def count_group(g, carry):
        sl = pl.ds(g * L, L)
        ones = [jnp.where(tok_v[l, sl] != 0, 1.0, 0.0)
                for l in range(NAME_LEN)]
        cnt = _treesum(ones)
        z_v[sl] = jnp.float32(NAME_LEN) - cnt
        rcnt_v[sl] = 1.0 / jnp.maximum(cnt, 1.0)
        return carry

    _s1.__exit__(None, None, None)
    _s2 = _jax.named_scope("p2_counts"); _s2.__enter__()
    lax.fori_loop(0, NGRP, count_group, 0)
    _s2.__exit__(None, None, None)

    def make_round(r):
        def round_body(g, carry):
            for k in range(L):
                i = g * L + k
                slabs = [namerows_v[(r * RND + j) * BPW + i, :]
                         for j in range(RND)]
                if r == 0:
                    acc = _treesum(slabs)
                else:
                    acc = nameacc_v[i, pl.ds(0, D_NAME)] + _treesum(slabs)
                nameacc_v[i, pl.ds(0, D_NAME)] = acc
            return carry
        return round_body

    for r in range(NROUND):
        _sw = _jax.named_scope(f"p3_wait{r}"); _sw.__enter__()
        for j in range(RND):
            name_copies[r * RND + j].wait()
        _sw.__exit__(None, None, None)
        _sp = _jax.named_scope(f"p4_round{r}"); _sp.__enter__()
        lax.fori_loop(0, NGRP, make_round(r), 0)
        _sp.__exit__(None, None, None)

    _s5 = _jax.named_scope("p5_finish"); _s5.__enter__()
    gics_stage.wait()

    def finish_group(g, carry):
        sl = pl.ds(g * L, L)
        items = iota + g * L
        z = z_v[sl]
        rc = rcnt_v[sl]
        for d in range(D_NAME):
            acc = plsc.load_gather(nameacc_v,
                                   [items, jnp.full((L,), d, jnp.int32)])
            out_tv[D_ID + d, sl] = (acc - z * emb0T_v[d, :]) * rc
        gidx = gicsidx_v[sl]
        for d in range(D_GICS):
            out_tv[D_ID + D_NAME + d, sl] = plsc.load_gather(
                gicstab_v, [gidx + d * GICS_VOCAB])
        return carry

    lax.fori_loop(0, NGRP, finish_group, 0)

    _s5.__exit__(None, None, None)
    _s6 = _jax.named_scope("p6_out"); _s6.__enter__()
    for cp in id_copies:
        cp.wait()
    pltpu.sync_copy(out_tv, out_hbm.at[:, pl.ds(base, BPW)])
    _s6.__exit__(None, None, None)


def kernel(emb_id, emb_name, emb_gics, item_id_idx, item_name_tokens,
           item_gics_idx):
    idv = item_id_idx.astype(jnp.int32)
    gicsv = item_gics_idx.astype(jnp.int32)
    tok_t = item_name_tokens.astype(jnp.int32).T
    emb_id_f = emb_id.T.reshape(-1)
    emb_gics_f = emb_gics.T.reshape(-1)
    mesh = plsc.VectorSubcoreMesh(core_axis_name="c", subcore_axis_name="s")
    f = pl.kernel(
        _body,
        out_type=jax.ShapeDtypeStruct((D_OUT, B), jnp.float32),
        mesh=mesh,
        compiler_params=pltpu.CompilerParams(
            needs_layout_passes=False, use_tc_tiling_on_sc=False),
        scratch_types=[
            pltpu.VMEM((BPW,), jnp.int32),
            pltpu.VMEM((BPW,), jnp.int32),
            pltpu.VMEM((D_ID, BPW), jnp.int32),
            pltpu.VMEM((NAME_LEN, BPW), jnp.int32),
            pltpu.VMEM((NAME_LEN * BPW, D_NAME), jnp.float32),
            pltpu.VMEM((D_GICS * GICS_VOCAB,), jnp.float32),
            pltpu.VMEM((BPW, D_NAME + 1), jnp.float32),
            pltpu.VMEM((1, D_NAME), jnp.float32),
            pltpu.VMEM((L, L), jnp.float32),
            pltpu.VMEM((BPW,), jnp.float32),
            pltpu.VMEM((BPW,), jnp.float32),
            pltpu.VMEM((D_OUT, BPW), jnp.float32),
            pltpu.SemaphoreType.DMA,
            pltpu.SemaphoreType.DMA,
            pltpu.SemaphoreType.DMA,
            pltpu.SemaphoreType.DMA,
        ] + [pltpu.SemaphoreType.DMA] * NROUND,
    )
    return f(emb_id_f, emb_name, emb_gics_f, idv, tok_t, gicsv).T


# speedup vs baseline: 1.1058x; 1.1058x over previous
import jax
import jax.numpy as jnp
from jax import lax
from jax.experimental import pallas as pl
from jax.experimental.pallas import tpu as pltpu
from jax.experimental.pallas import tpu_sc as plsc

B = 4096
NAME_LEN = 20
D_ID = 8
D_NAME = 16
D_GICS = 8
D_OUT = D_ID + D_NAME + D_GICS
ID_VOCAB = 100001
GICS_VOCAB = 1001

NC, NS, L = 2, 16, 16
NW = NC * NS
BPW = B // NW
NGRP = BPW // L
RND = 4
NROUND = NAME_LEN // RND


def _treesum(vs):
    while len(vs) > 1:
        vs = [a + b for a, b in zip(vs[::2], vs[1::2])] + (
            [vs[-1]] if len(vs) % 2 else [])
    return vs[0]


def _body(emb_id_f, emb_name, emb_gics_f, idv, tok_t, gicsv, out_hbm,
          ididx_v, gicsidx_v, idix_v, tok_v, namerows_v, gicstab_v,
          nameacc_v, emb0_v, emb0T_v, z_v, rcnt_v, out_tv,
          semid, semg, sem2, semt, *semn):
    wid = lax.axis_index("c") * NS + lax.axis_index("s")
    base = wid * BPW
    iota = lax.iota(jnp.int32, L)

    gics_stage = pltpu.async_copy(emb_gics_f, gicstab_v, semg)
    tok_stage = pltpu.async_copy(tok_t.at[:, pl.ds(base, BPW)], tok_v, semt)
    stage = [
        pltpu.async_copy(idv.at[pl.ds(base, BPW)], ididx_v, sem2),
        pltpu.async_copy(gicsv.at[pl.ds(base, BPW)], gicsidx_v, sem2),
        pltpu.async_copy(emb_name.at[pl.ds(0, 1)], emb0_v, sem2),
    ]

    tok_stage.wait()
    name_copies = [pltpu.async_copy(emb_name.at[tok_v.at[l]],
                                    namerows_v.at[pl.ds(l * BPW, BPW)],
                                    semn[l // RND])
                   for l in range(NAME_LEN)]

    for cp in stage:
        cp.wait()

    for d in range(D_ID):
        for c in range(NGRP):
            sl = pl.ds(c * L, L)
            idix_v[d, sl] = ididx_v[sl] + d * ID_VOCAB
    id_copies = [pltpu.async_copy(emb_id_f.at[idix_v.at[d]],
                                  out_tv.at[d], semid)
                 for d in range(D_ID)]

    e0vec = emb0_v[0, :]
    for c in range(L):
        plsc.store_scatter(emb0T_v, [iota, jnp.full((L,), c, jnp.int32)],
                           e0vec)

    def count_group(g, carry):
        sl = pl.ds(g * L, L)
        ones = [jnp.where(tok_v[l, sl] != 0, 1.0, 0.0)
                for l in range(NAME_LEN)]
        cnt = _treesum(ones)
        z_v[sl] = jnp.float32(NAME_LEN) - cnt
        rcnt_v[sl] = 1.0 / jnp.maximum(cnt, 1.0)
        return carry

    lax.fori_loop(0, NGRP, count_group, 0)

    def make_round(r):
        def round_body(g, carry):
            for k in range(L):
                i = g * L + k
                slabs = [namerows_v[(r * RND + j) * BPW + i, :]
                         for j in range(RND)]
                if r == 0:
                    acc = _treesum(slabs)
                else:
                    acc = nameacc_v[i, :] + _treesum(slabs)
                nameacc_v[i, :] = acc
            return carry
        return round_body

    for r in range(NROUND):
        for j in range(RND):
            name_copies[r * RND + j].wait()
        lax.fori_loop(0, NGRP, make_round(r), 0)

    gics_stage.wait()
    colvs = [(iota + t) & (L - 1) for t in range(L)]
    e0d = [plsc.load_gather(emb0T_v, [colvs[t], iota]) for t in range(L)]

    def finish_group(g, carry):
        sl = pl.ds(g * L, L)
        items = iota + g * L
        z = z_v[sl]
        rc = rcnt_v[sl]
        for t in range(L):
            acc = plsc.load_gather(nameacc_v, [items, colvs[t]])
            plsc.store_scatter(out_tv, [colvs[t] + D_ID, items],
                               (acc - z * e0d[t]) * rc)
        gidx = gicsidx_v[sl]
        for d in range(D_GICS):
            out_tv[D_ID + D_NAME + d, sl] = plsc.load_gather(
                gicstab_v, [gidx + d * GICS_VOCAB])
        return carry

    lax.fori_loop(0, NGRP, finish_group, 0)

    for cp in id_copies:
        cp.wait()
    pltpu.sync_copy(out_tv, out_hbm.at[:, pl.ds(base, BPW)])


def kernel(emb_id, emb_name, emb_gics, item_id_idx, item_name_tokens,
           item_gics_idx):
    idv = item_id_idx.astype(jnp.int32)
    gicsv = item_gics_idx.astype(jnp.int32)
    tok_t = item_name_tokens.astype(jnp.int32).T
    emb_id_f = emb_id.T.reshape(-1)
    emb_gics_f = emb_gics.T.reshape(-1)
    mesh = plsc.VectorSubcoreMesh(core_axis_name="c", subcore_axis_name="s")
    f = pl.kernel(
        _body,
        out_type=jax.ShapeDtypeStruct((D_OUT, B), jnp.float32),
        mesh=mesh,
        compiler_params=pltpu.CompilerParams(
            needs_layout_passes=False, use_tc_tiling_on_sc=False),
        scratch_types=[
            pltpu.VMEM((BPW,), jnp.int32),
            pltpu.VMEM((BPW,), jnp.int32),
            pltpu.VMEM((D_ID, BPW), jnp.int32),
            pltpu.VMEM((NAME_LEN, BPW), jnp.int32),
            pltpu.VMEM((NAME_LEN * BPW, D_NAME), jnp.float32),
            pltpu.VMEM((D_GICS * GICS_VOCAB,), jnp.float32),
            pltpu.VMEM((BPW, D_NAME), jnp.float32),
            pltpu.VMEM((1, D_NAME), jnp.float32),
            pltpu.VMEM((L, L), jnp.float32),
            pltpu.VMEM((BPW,), jnp.float32),
            pltpu.VMEM((BPW,), jnp.float32),
            pltpu.VMEM((D_OUT, BPW), jnp.float32),
            pltpu.SemaphoreType.DMA,
            pltpu.SemaphoreType.DMA,
            pltpu.SemaphoreType.DMA,
            pltpu.SemaphoreType.DMA,
        ] + [pltpu.SemaphoreType.DMA] * NROUND,
    )
    return f(emb_id_f, emb_name, emb_gics_f, idv, tok_t, gicsv).T
